# Initial kernel scaffold; baseline (speedup 1.0000x reference)
import jax
import jax.numpy as jnp
from jax.experimental import pallas as pl


def kernel(x, adj, batch, W1, a1_src, a1_dst, b1, W2, a2_src, a2_dst, b2, linW, linb):
    raise NotImplementedError("write your pallas kernel here")


# speedup vs baseline: 21.3463x; 21.3463x over previous
import functools

import jax
import jax.numpy as jnp
from jax import lax
from jax.experimental import pallas as pl
from jax.experimental.pallas import tpu as pltpu
from jax.experimental.pallas import tpu_sc as plsc

_N = 10000
_E = 320000
_D = 128
_HID = 128
_OUT = 64
_G = 128

_NC, _NS, _L = 2, 16, 16
_CHUNK = 128
_ETOT = _E + _N
_NCHUNK = -(-_ETOT // (_NS * _CHUNK))
_EPAD = _NS * _NCHUNK * _CHUNK
_NPAD = 10240
_STRIPE = _NPAD // _NS
_ROWBLK = 1024


def _edge_kernel_body(Df, hs_hbm, as_hbm, ad_hbm, src_hbm, dst_hbm,
                      num_out, den_out,
                      as_l, ad_l, src_l, dst_l, w_l, rows_l,
                      num_s, den_s, sem):
    half = Df // 2
    cid = lax.axis_index("c")
    sid = lax.axis_index("s")

    pltpu.sync_copy(as_hbm, as_l)
    pltpu.sync_copy(ad_hbm, ad_l)
    pltpu.sync_copy(src_hbm.at[sid], src_l)
    pltpu.sync_copy(dst_hbm.at[sid], dst_l)

    def _maxstep_a(i, m):
        return jnp.maximum(m, as_l[pl.ds(i * _L, _L)])
    def _maxstep_b(i, m):
        return jnp.maximum(m, ad_l[pl.ds(i * _L, _L)])
    neg = jnp.full((_L,), -3e38, jnp.float32)

    def _lane_max_splat(v16):
        w_l[pl.ds(0, _L)] = v16
        acc = plsc.load_gather(w_l, [jnp.zeros((_L,), jnp.int32)])
        for k in range(1, _L):
            acc = jnp.maximum(
                acc, plsc.load_gather(w_l, [jnp.full((_L,), k, jnp.int32)]))
        return acc

    masv = _lane_max_splat(lax.fori_loop(0, _NPAD // _L, _maxstep_a, neg))
    madv = _lane_max_splat(lax.fori_loop(0, _NPAD // _L, _maxstep_b, neg))
    msum = masv + madv
    cbound = jnp.maximum(msum, 0.2 * msum)

    zero16 = jnp.zeros((_L,), jnp.float32)
    def _zrow(i, _):
        for f in range(half // _L):
            rows_l[i, pl.ds(f * _L, _L)] = zero16
        return 0
    lax.fori_loop(0, _CHUNK, _zrow, 0)
    for k in range(_CHUNK // _L):
        w_l[pl.ds(k * _L, _L)] = zero16
    base = sid * _STRIPE
    for t in range(_STRIPE // _CHUNK):
        pltpu.sync_copy(rows_l, num_s.at[pl.ds(base + t * _CHUNK, _CHUNK)])
        pltpu.sync_copy(w_l, den_s.at[pl.ds(base + t * _CHUNK, _CHUNK)])
    plsc.subcore_barrier()

    def _chunk(j, _):
        pltpu.async_copy(hs_hbm.at[cid].at[src_l.at[j]], rows_l, sem).wait()
        for k in range(_CHUNK // _L):
            si = src_l[j, pl.ds(k * _L, _L)]
            di = dst_l[j, pl.ds(k * _L, _L)]
            s = (plsc.load_gather(as_l, [si]) +
                 plsc.load_gather(ad_l, [di]))
            e = jnp.maximum(s, 0.2 * s)
            w_l[pl.ds(k * _L, _L)] = jnp.exp(e - cbound)
        def _scale(i, _):
            wv = plsc.load_gather(
                w_l, [jnp.broadcast_to(i, (_L,)).astype(jnp.int32)])
            for f in range(half // _L):
                sl = pl.ds(f * _L, _L)
                rows_l[i, sl] = rows_l[i, sl] * wv
            return 0
        lax.fori_loop(0, _CHUNK, _scale, 0)
        pltpu.sync_copy(rows_l, num_s.at[dst_l.at[j]], add=True)
        pltpu.sync_copy(w_l, den_s.at[dst_l.at[j]], add=True)
        return 0
    lax.fori_loop(0, _NCHUNK, _chunk, 0)
    plsc.subcore_barrier()

    pltpu.sync_copy(num_s.at[pl.ds(base, _STRIPE)],
                    num_out.at[cid, pl.ds(base, _STRIPE)])
    @pl.when(cid == 0)
    def _():
        pltpu.sync_copy(den_s.at[pl.ds(base, _STRIPE)],
                        den_out.at[pl.ds(base, _STRIPE)])


def _make_edge_kernel(Df):
    half = Df // 2
    mesh = plsc.VectorSubcoreMesh(core_axis_name="c", subcore_axis_name="s")
    return pl.kernel(
        functools.partial(_edge_kernel_body, Df),
        out_type=(jax.ShapeDtypeStruct((_NC, _NPAD, half), jnp.float32),
                  jax.ShapeDtypeStruct((_NPAD,), jnp.float32)),
        mesh=mesh,
        scratch_types=(
            pltpu.VMEM((_NPAD,), jnp.float32),
            pltpu.VMEM((_NPAD,), jnp.float32),
            pltpu.VMEM((_NCHUNK, _CHUNK), jnp.int32),
            pltpu.VMEM((_NCHUNK, _CHUNK), jnp.int32),
            pltpu.VMEM((_CHUNK,), jnp.float32),
            pltpu.VMEM((_CHUNK, half), jnp.float32),
            pltpu.VMEM_SHARED((_NPAD, half), jnp.float32),
            pltpu.VMEM_SHARED((_NPAD,), jnp.float32),
            pltpu.SemaphoreType.DMA,
        ),
        compiler_params=pltpu.CompilerParams(
            needs_layout_passes=False, use_tc_tiling_on_sc=False),
        name=f"gat_edge_sc_{Df}",
    )


def _tc_a_body(x_ref, w_ref, asr, adr, h_ref, sa_ref, da_ref):
    h = jnp.dot(x_ref[...], w_ref[...], preferred_element_type=jnp.float32)
    h_ref[...] = h
    sa_ref[...] = (h * asr[...]).sum(-1, keepdims=True)
    da_ref[...] = (h * adr[...]).sum(-1, keepdims=True)


def _tc_b_body(num_ref, den_ref, b1_ref, w2_ref, asr, adr,
               h2_ref, sa_ref, da_ref):
    d = den_ref[...] + 1e-30
    h = jax.nn.relu(num_ref[...] / d + b1_ref[...])
    h2 = jnp.dot(h, w2_ref[...], preferred_element_type=jnp.float32)
    h2_ref[...] = h2
    sa_ref[...] = (h2 * asr[...]).sum(-1, keepdims=True)
    da_ref[...] = (h2 * adr[...]).sum(-1, keepdims=True)


def _tc_c_body(num_ref, den_ref, b2_ref, boh_ref, lw_ref, lb_ref,
               res_ref, acc_ref):
    i = pl.program_id(0)
    @pl.when(i == 0)
    def _():
        acc_ref[...] = jnp.zeros_like(acc_ref)
    d = den_ref[...] + 1e-30
    o = num_ref[...] / d + b2_ref[...]
    z = o - o.max(-1, keepdims=True)
    ls = z - jnp.log(jnp.exp(z).sum(-1, keepdims=True))
    lsa = jnp.concatenate([ls, jnp.ones_like(ls)], axis=-1)
    acc_ref[...] += lax.dot_general(
        boh_ref[...], lsa, (((0,), (0,)), ((), ())),
        preferred_element_type=jnp.float32)
    @pl.when(i == pl.num_programs(0) - 1)
    def _():
        acc = acc_ref[...]
        pm = acc[:, :_OUT] / jnp.maximum(acc[:, _OUT:_OUT + 1], 1.0)
        res_ref[...] = jnp.dot(pm, lw_ref[...],
                               preferred_element_type=jnp.float32) + lb_ref[...]


_NB = _NPAD // _ROWBLK


def _tc_a(xp, W1, a1s, a1d):
    return pl.pallas_call(
        _tc_a_body,
        grid=(_NB,),
        in_specs=[
            pl.BlockSpec((_ROWBLK, _D), lambda i: (i, 0)),
            pl.BlockSpec((_D, _HID), lambda i: (0, 0)),
            pl.BlockSpec((1, _HID), lambda i: (0, 0)),
            pl.BlockSpec((1, _HID), lambda i: (0, 0)),
        ],
        out_specs=[
            pl.BlockSpec((_ROWBLK, _HID), lambda i: (i, 0)),
            pl.BlockSpec((_ROWBLK, 1), lambda i: (i, 0)),
            pl.BlockSpec((_ROWBLK, 1), lambda i: (i, 0)),
        ],
        out_shape=[
            jax.ShapeDtypeStruct((_NPAD, _HID), jnp.float32),
            jax.ShapeDtypeStruct((_NPAD, 1), jnp.float32),
            jax.ShapeDtypeStruct((_NPAD, 1), jnp.float32),
        ],
    )(xp, W1, a1s, a1d)


def _tc_b(num1, den1, b1, W2, a2s, a2d):
    return pl.pallas_call(
        _tc_b_body,
        grid=(_NB,),
        in_specs=[
            pl.BlockSpec((_ROWBLK, _HID), lambda i: (i, 0)),
            pl.BlockSpec((_ROWBLK, 1), lambda i: (i, 0)),
            pl.BlockSpec((1, _HID), lambda i: (0, 0)),
            pl.BlockSpec((_HID, _OUT), lambda i: (0, 0)),
            pl.BlockSpec((1, _OUT), lambda i: (0, 0)),
            pl.BlockSpec((1, _OUT), lambda i: (0, 0)),
        ],
        out_specs=[
            pl.BlockSpec((_ROWBLK, _OUT), lambda i: (i, 0)),
            pl.BlockSpec((_ROWBLK, 1), lambda i: (i, 0)),
            pl.BlockSpec((_ROWBLK, 1), lambda i: (i, 0)),
        ],
        out_shape=[
            jax.ShapeDtypeStruct((_NPAD, _OUT), jnp.float32),
            jax.ShapeDtypeStruct((_NPAD, 1), jnp.float32),
            jax.ShapeDtypeStruct((_NPAD, 1), jnp.float32),
        ],
    )(num1, den1, b1, W2, a2s, a2d)


def _tc_c(num2, den2, b2, boh, lwp, lbp):
    return pl.pallas_call(
        _tc_c_body,
        grid=(_NB,),
        in_specs=[
            pl.BlockSpec((_ROWBLK, _OUT), lambda i: (i, 0)),
            pl.BlockSpec((_ROWBLK, 1), lambda i: (i, 0)),
            pl.BlockSpec((1, _OUT), lambda i: (0, 0)),
            pl.BlockSpec((_ROWBLK, _G), lambda i: (i, 0)),
            pl.BlockSpec((_OUT, _G), lambda i: (0, 0)),
            pl.BlockSpec((1, _G), lambda i: (0, 0)),
        ],
        out_specs=pl.BlockSpec((_G, _G), lambda i: (0, 0)),
        out_shape=jax.ShapeDtypeStruct((_G, _G), jnp.float32),
        scratch_shapes=[pltpu.VMEM((_G, _G), jnp.float32)],
    )(num2, den2, b2, boh, lwp, lbp)


def _split_cols(h, Df):
    half = Df // 2
    return jnp.stack([h[:, :half], h[:, half:]], axis=0)


def kernel(x, adj, batch, W1, a1_src, a1_dst, b1, W2, a2_src, a2_dst, b2,
           linW, linb):
    xp = jnp.pad(x, ((0, _NPAD - _N), (0, 0)))
    loops = jnp.arange(_N, dtype=jnp.int32)
    src_all = jnp.concatenate(
        [adj[0], loops, jnp.zeros((_EPAD - _ETOT,), jnp.int32)])
    dst_all = jnp.concatenate(
        [adj[1], loops, jnp.full((_EPAD - _ETOT,), _N, jnp.int32)])
    srcg = src_all.reshape(_NS, _NCHUNK, _CHUNK)
    dstg = dst_all.reshape(_NS, _NCHUNK, _CHUNK)
    batch_pad = jnp.concatenate(
        [batch, jnp.full((_NPAD - _N,), _G, jnp.int32)])
    boh = jax.nn.one_hot(batch_pad, _G, dtype=jnp.float32)
    lwp = jnp.pad(linW, ((0, 0), (0, _G - 1)))
    lbp = jnp.broadcast_to(linb.reshape(1, 1), (1, _G))
    b1r = b1.reshape(1, _HID)
    b2r = b2.reshape(1, _OUT)

    h1, as1, ad1 = _tc_a(xp, W1, a1_src, a1_dst)
    ek1 = _make_edge_kernel(_HID)
    num1p, den1 = ek1(_split_cols(h1, _HID), as1.reshape(_NPAD),
                      ad1.reshape(_NPAD), srcg, dstg)
    num1 = jnp.concatenate([num1p[0], num1p[1]], axis=1)

    h2, as2, ad2 = _tc_b(num1, den1.reshape(_NPAD, 1), b1r, W2,
                         a2_src, a2_dst)
    ek2 = _make_edge_kernel(_OUT)
    num2p, den2 = ek2(_split_cols(h2, _OUT), as2.reshape(_NPAD),
                      ad2.reshape(_NPAD), srcg, dstg)
    num2 = jnp.concatenate([num2p[0], num2p[1]], axis=1)

    res = _tc_c(num2, den2.reshape(_NPAD, 1), b2r, boh, lwp, lbp)
    return res[:, :1]


# speedup vs baseline: 23.0972x; 1.0820x over previous
import functools

import jax
import jax.numpy as jnp
from jax import lax
from jax.experimental import pallas as pl
from jax.experimental.pallas import tpu as pltpu
from jax.experimental.pallas import tpu_sc as plsc

_N = 10000
_E = 320000
_D = 128
_HID = 128
_OUT = 64
_G = 128

_NC, _NS, _L = 2, 16, 16
_CHUNK = 128
_ETOT = _E + _N
_NCHUNK = -(-_ETOT // (_NS * _CHUNK))
_EPAD = _NS * _NCHUNK * _CHUNK
_NPAD = 10240
_STRIPE = _NPAD // _NS
_ROWBLK = 1024


def _edge_kernel_body(Df, hs_hbm, as_hbm, ad_hbm, src_hbm, dst_hbm,
                      num_out, den_out,
                      as_l, ad_l, src_l, dst_l,
                      w0_l, w1_l, rows0_l, rows1_l,
                      num_s, den_s, gsem0, gsem1, ssem0, ssem1):
    half = Df // 2
    cid = lax.axis_index("c")
    sid = lax.axis_index("s")
    w_l = w0_l

    pltpu.sync_copy(as_hbm, as_l)
    pltpu.sync_copy(ad_hbm, ad_l)
    pltpu.sync_copy(src_hbm.at[sid], src_l)
    pltpu.sync_copy(dst_hbm.at[sid], dst_l)

    def _maxstep_a(i, m):
        return jnp.maximum(m, as_l[pl.ds(i * _L, _L)])
    def _maxstep_b(i, m):
        return jnp.maximum(m, ad_l[pl.ds(i * _L, _L)])
    neg = jnp.full((_L,), -3e38, jnp.float32)

    def _lane_max_splat(v16):
        w_l[pl.ds(0, _L)] = v16
        acc = plsc.load_gather(w_l, [jnp.zeros((_L,), jnp.int32)])
        for k in range(1, _L):
            acc = jnp.maximum(
                acc, plsc.load_gather(w_l, [jnp.full((_L,), k, jnp.int32)]))
        return acc

    masv = _lane_max_splat(lax.fori_loop(0, _NPAD // _L, _maxstep_a, neg))
    madv = _lane_max_splat(lax.fori_loop(0, _NPAD // _L, _maxstep_b, neg))
    msum = masv + madv
    cbound = jnp.maximum(msum, 0.2 * msum)

    zero16 = jnp.zeros((_L,), jnp.float32)
    def _zrow(i, _):
        for f in range(half // _L):
            rows0_l[i, pl.ds(f * _L, _L)] = zero16
            rows1_l[i, pl.ds(f * _L, _L)] = zero16
        return 0
    lax.fori_loop(0, _CHUNK, _zrow, 0)
    for k in range(_CHUNK // _L):
        w0_l[pl.ds(k * _L, _L)] = zero16
        w1_l[pl.ds(k * _L, _L)] = zero16
    base = sid * _STRIPE
    for t in range(_STRIPE // _CHUNK):
        pltpu.sync_copy(rows0_l, num_s.at[pl.ds(base + t * _CHUNK, _CHUNK)])
        pltpu.sync_copy(w0_l, den_s.at[pl.ds(base + t * _CHUNK, _CHUNK)])
    plsc.subcore_barrier()

    def _compute(j, wbuf, rows):
        for k in range(_CHUNK // _L):
            si = src_l[j, pl.ds(k * _L, _L)]
            di = dst_l[j, pl.ds(k * _L, _L)]
            s = (plsc.load_gather(as_l, [si]) +
                 plsc.load_gather(ad_l, [di]))
            e = jnp.maximum(s, 0.2 * s)
            wbuf[pl.ds(k * _L, _L)] = jnp.exp(e - cbound)
        for i in range(_CHUNK):
            wv = plsc.load_gather(wbuf, [jnp.full((_L,), i, jnp.int32)])
            for f in range(half // _L):
                sl = pl.ds(f * _L, _L)
                rows[i, sl] = rows[i, sl] * wv

    def _issue_gather(j, rows, gsem):
        return pltpu.async_copy(hs_hbm.at[cid].at[src_l.at[j]], rows, gsem)

    def _issue_scatter(j, wbuf, rows, ssem):
        pltpu.async_copy(rows, num_s.at[dst_l.at[j]], ssem, add=True)
        pltpu.async_copy(wbuf, den_s.at[dst_l.at[j]], ssem, add=True)

    def _drain_gather(rows, gsem):
        pltpu.make_async_copy(hs_hbm.at[cid, pl.ds(0, _CHUNK)],
                              rows, gsem).wait()

    def _drain_scatter(wbuf, rows, ssem):
        pltpu.make_async_copy(hs_hbm.at[cid, pl.ds(0, _CHUNK)],
                              rows, ssem).wait()
        pltpu.make_async_copy(as_hbm.at[pl.ds(0, _CHUNK)],
                              wbuf, ssem).wait()

    _issue_scatter(0, w0_l, rows0_l, ssem0)
    _issue_scatter(0, w1_l, rows1_l, ssem1)
    _drain_scatter(w0_l, rows0_l, ssem0)
    _issue_gather(0, rows0_l, gsem0)

    def _pair(k, _):
        a = 2 * k
        b = a + 1
        _drain_gather(rows0_l, gsem0)
        _drain_scatter(w1_l, rows1_l, ssem1)
        _issue_gather(b, rows1_l, gsem1)
        _compute(a, w0_l, rows0_l)
        _issue_scatter(a, w0_l, rows0_l, ssem0)
        _drain_gather(rows1_l, gsem1)
        _compute(b, w1_l, rows1_l)
        _drain_scatter(w0_l, rows0_l, ssem0)
        _issue_gather(jnp.minimum(a + 2, _NCHUNK - 1), rows0_l, gsem0)
        _issue_scatter(b, w1_l, rows1_l, ssem1)
        return 0
    lax.fori_loop(0, _NCHUNK // 2, _pair, 0)
    _drain_gather(rows0_l, gsem0)
    _drain_scatter(w1_l, rows1_l, ssem1)
    plsc.subcore_barrier()

    pltpu.sync_copy(num_s.at[pl.ds(base, _STRIPE)],
                    num_out.at[cid, pl.ds(base, _STRIPE)])
    @pl.when(cid == 0)
    def _():
        pltpu.sync_copy(den_s.at[pl.ds(base, _STRIPE)],
                        den_out.at[pl.ds(base, _STRIPE)])


def _make_edge_kernel(Df):
    half = Df // 2
    mesh = plsc.VectorSubcoreMesh(core_axis_name="c", subcore_axis_name="s")
    return pl.kernel(
        functools.partial(_edge_kernel_body, Df),
        out_type=(jax.ShapeDtypeStruct((_NC, _NPAD, half), jnp.float32),
                  jax.ShapeDtypeStruct((_NPAD,), jnp.float32)),
        mesh=mesh,
        scratch_types=(
            pltpu.VMEM((_NPAD,), jnp.float32),
            pltpu.VMEM((_NPAD,), jnp.float32),
            pltpu.VMEM((_NCHUNK, _CHUNK), jnp.int32),
            pltpu.VMEM((_NCHUNK, _CHUNK), jnp.int32),
            pltpu.VMEM((_CHUNK,), jnp.float32),
            pltpu.VMEM((_CHUNK,), jnp.float32),
            pltpu.VMEM((_CHUNK, half), jnp.float32),
            pltpu.VMEM((_CHUNK, half), jnp.float32),
            pltpu.VMEM_SHARED((_NPAD, half), jnp.float32),
            pltpu.VMEM_SHARED((_NPAD,), jnp.float32),
            pltpu.SemaphoreType.DMA,
            pltpu.SemaphoreType.DMA,
            pltpu.SemaphoreType.DMA,
            pltpu.SemaphoreType.DMA,
        ),
        compiler_params=pltpu.CompilerParams(
            needs_layout_passes=False, use_tc_tiling_on_sc=False),
        name=f"gat_edge_sc_{Df}",
    )


def _tc_a_body(x_ref, w_ref, asr, adr, h_ref, sa_ref, da_ref):
    h = jnp.dot(x_ref[...], w_ref[...], preferred_element_type=jnp.float32)
    h_ref[...] = h
    sa_ref[...] = (h * asr[...]).sum(-1, keepdims=True)
    da_ref[...] = (h * adr[...]).sum(-1, keepdims=True)


def _tc_b_body(num_ref, den_ref, b1_ref, w2_ref, asr, adr,
               h2_ref, sa_ref, da_ref):
    d = den_ref[...] + 1e-30
    h = jax.nn.relu(num_ref[...] / d + b1_ref[...])
    h2 = jnp.dot(h, w2_ref[...], preferred_element_type=jnp.float32)
    h2_ref[...] = h2
    sa_ref[...] = (h2 * asr[...]).sum(-1, keepdims=True)
    da_ref[...] = (h2 * adr[...]).sum(-1, keepdims=True)


def _tc_c_body(num_ref, den_ref, b2_ref, boh_ref, lw_ref, lb_ref,
               res_ref, acc_ref):
    i = pl.program_id(0)
    @pl.when(i == 0)
    def _():
        acc_ref[...] = jnp.zeros_like(acc_ref)
    d = den_ref[...] + 1e-30
    o = num_ref[...] / d + b2_ref[...]
    z = o - o.max(-1, keepdims=True)
    ls = z - jnp.log(jnp.exp(z).sum(-1, keepdims=True))
    lsa = jnp.concatenate([ls, jnp.ones_like(ls)], axis=-1)
    acc_ref[...] += lax.dot_general(
        boh_ref[...], lsa, (((0,), (0,)), ((), ())),
        preferred_element_type=jnp.float32)
    @pl.when(i == pl.num_programs(0) - 1)
    def _():
        acc = acc_ref[...]
        pm = acc[:, :_OUT] / jnp.maximum(acc[:, _OUT:_OUT + 1], 1.0)
        res_ref[...] = jnp.dot(pm, lw_ref[...],
                               preferred_element_type=jnp.float32) + lb_ref[...]


_NB = _NPAD // _ROWBLK


def _tc_a(xp, W1, a1s, a1d):
    return pl.pallas_call(
        _tc_a_body,
        grid=(_NB,),
        in_specs=[
            pl.BlockSpec((_ROWBLK, _D), lambda i: (i, 0)),
            pl.BlockSpec((_D, _HID), lambda i: (0, 0)),
            pl.BlockSpec((1, _HID), lambda i: (0, 0)),
            pl.BlockSpec((1, _HID), lambda i: (0, 0)),
        ],
        out_specs=[
            pl.BlockSpec((_ROWBLK, _HID), lambda i: (i, 0)),
            pl.BlockSpec((_ROWBLK, 1), lambda i: (i, 0)),
            pl.BlockSpec((_ROWBLK, 1), lambda i: (i, 0)),
        ],
        out_shape=[
            jax.ShapeDtypeStruct((_NPAD, _HID), jnp.float32),
            jax.ShapeDtypeStruct((_NPAD, 1), jnp.float32),
            jax.ShapeDtypeStruct((_NPAD, 1), jnp.float32),
        ],
    )(xp, W1, a1s, a1d)


def _tc_b(num1, den1, b1, W2, a2s, a2d):
    return pl.pallas_call(
        _tc_b_body,
        grid=(_NB,),
        in_specs=[
            pl.BlockSpec((_ROWBLK, _HID), lambda i: (i, 0)),
            pl.BlockSpec((_ROWBLK, 1), lambda i: (i, 0)),
            pl.BlockSpec((1, _HID), lambda i: (0, 0)),
            pl.BlockSpec((_HID, _OUT), lambda i: (0, 0)),
            pl.BlockSpec((1, _OUT), lambda i: (0, 0)),
            pl.BlockSpec((1, _OUT), lambda i: (0, 0)),
        ],
        out_specs=[
            pl.BlockSpec((_ROWBLK, _OUT), lambda i: (i, 0)),
            pl.BlockSpec((_ROWBLK, 1), lambda i: (i, 0)),
            pl.BlockSpec((_ROWBLK, 1), lambda i: (i, 0)),
        ],
        out_shape=[
            jax.ShapeDtypeStruct((_NPAD, _OUT), jnp.float32),
            jax.ShapeDtypeStruct((_NPAD, 1), jnp.float32),
            jax.ShapeDtypeStruct((_NPAD, 1), jnp.float32),
        ],
    )(num1, den1, b1, W2, a2s, a2d)


def _tc_c(num2, den2, b2, boh, lwp, lbp):
    return pl.pallas_call(
        _tc_c_body,
        grid=(_NB,),
        in_specs=[
            pl.BlockSpec((_ROWBLK, _OUT), lambda i: (i, 0)),
            pl.BlockSpec((_ROWBLK, 1), lambda i: (i, 0)),
            pl.BlockSpec((1, _OUT), lambda i: (0, 0)),
            pl.BlockSpec((_ROWBLK, _G), lambda i: (i, 0)),
            pl.BlockSpec((_OUT, _G), lambda i: (0, 0)),
            pl.BlockSpec((1, _G), lambda i: (0, 0)),
        ],
        out_specs=pl.BlockSpec((_G, _G), lambda i: (0, 0)),
        out_shape=jax.ShapeDtypeStruct((_G, _G), jnp.float32),
        scratch_shapes=[pltpu.VMEM((_G, _G), jnp.float32)],
    )(num2, den2, b2, boh, lwp, lbp)


def _split_cols(h, Df):
    half = Df // 2
    return jnp.stack([h[:, :half], h[:, half:]], axis=0)


def kernel(x, adj, batch, W1, a1_src, a1_dst, b1, W2, a2_src, a2_dst, b2,
           linW, linb):
    xp = jnp.pad(x, ((0, _NPAD - _N), (0, 0)))
    loops = jnp.arange(_N, dtype=jnp.int32)
    src_all = jnp.concatenate(
        [adj[0], loops, jnp.zeros((_EPAD - _ETOT,), jnp.int32)])
    dst_all = jnp.concatenate(
        [adj[1], loops, jnp.full((_EPAD - _ETOT,), _N, jnp.int32)])
    srcg = src_all.reshape(_NS, _NCHUNK, _CHUNK)
    dstg = dst_all.reshape(_NS, _NCHUNK, _CHUNK)
    batch_pad = jnp.concatenate(
        [batch, jnp.full((_NPAD - _N,), _G, jnp.int32)])
    boh = jax.nn.one_hot(batch_pad, _G, dtype=jnp.float32)
    lwp = jnp.pad(linW, ((0, 0), (0, _G - 1)))
    lbp = jnp.broadcast_to(linb.reshape(1, 1), (1, _G))
    b1r = b1.reshape(1, _HID)
    b2r = b2.reshape(1, _OUT)

    h1, as1, ad1 = _tc_a(xp, W1, a1_src, a1_dst)
    ek1 = _make_edge_kernel(_HID)
    num1p, den1 = ek1(_split_cols(h1, _HID), as1.reshape(_NPAD),
                      ad1.reshape(_NPAD), srcg, dstg)
    num1 = jnp.concatenate([num1p[0], num1p[1]], axis=1)

    h2, as2, ad2 = _tc_b(num1, den1.reshape(_NPAD, 1), b1r, W2,
                         a2_src, a2_dst)
    ek2 = _make_edge_kernel(_OUT)
    num2p, den2 = ek2(_split_cols(h2, _OUT), as2.reshape(_NPAD),
                      ad2.reshape(_NPAD), srcg, dstg)
    num2 = jnp.concatenate([num2p[0], num2p[1]], axis=1)

    res = _tc_c(num2, den2.reshape(_NPAD, 1), b2r, boh, lwp, lbp)
    return res[:, :1]


# speedup vs baseline: 34.2396x; 1.4824x over previous
import functools

import jax
import jax.numpy as jnp
from jax import lax
from jax.experimental import pallas as pl
from jax.experimental.pallas import tpu as pltpu
from jax.experimental.pallas import tpu_sc as plsc

_N = 10000
_E = 320000
_D = 128
_HID = 128
_OUT = 64
_G = 128

_NC, _NS, _L = 2, 16, 16
_CHUNK = 128
_ETOT = _E + _N
_NCHUNK = -(-_ETOT // (_NS * _CHUNK))
_EPAD = _NS * _NCHUNK * _CHUNK
_NPAD = 10240
_STRIPE = _NPAD // _NS
_ROWBLK = 1024


def _edge_kernel_body(Df, hs_hbm, as_hbm, ad_hbm, src_hbm, dst_hbm,
                      num_out, den_out,
                      as_l, ad_l, src_l, dst_l,
                      w0_l, w1_l, rows0_l, rows1_l,
                      num_s, den_s, gsem0, gsem1, ssem0, ssem1):
    half = Df // 2
    cid = lax.axis_index("c")
    sid = lax.axis_index("s")
    w_l = w0_l

    pltpu.sync_copy(as_hbm, as_l)
    pltpu.sync_copy(ad_hbm, ad_l)
    pltpu.sync_copy(src_hbm.at[sid], src_l)
    pltpu.sync_copy(dst_hbm.at[sid], dst_l)

    def _maxstep_a(i, m):
        return jnp.maximum(m, as_l[pl.ds(i * _L, _L)])
    def _maxstep_b(i, m):
        return jnp.maximum(m, ad_l[pl.ds(i * _L, _L)])
    neg = jnp.full((_L,), -3e38, jnp.float32)

    def _lane_max_splat(v16):
        w_l[pl.ds(0, _L)] = v16
        acc = plsc.load_gather(w_l, [jnp.zeros((_L,), jnp.int32)])
        for k in range(1, _L):
            acc = jnp.maximum(
                acc, plsc.load_gather(w_l, [jnp.full((_L,), k, jnp.int32)]))
        return acc

    masv = _lane_max_splat(lax.fori_loop(0, _NPAD // _L, _maxstep_a, neg))
    madv = _lane_max_splat(lax.fori_loop(0, _NPAD // _L, _maxstep_b, neg))
    msum = masv + madv
    cbound = jnp.maximum(msum, 0.2 * msum)

    zero16 = jnp.zeros((_L,), jnp.float32)
    def _zrow(i, _):
        for f in range(half // _L):
            rows0_l[i, pl.ds(f * _L, _L)] = zero16
            rows1_l[i, pl.ds(f * _L, _L)] = zero16
        return 0
    lax.fori_loop(0, _CHUNK, _zrow, 0)
    for k in range(_CHUNK // _L):
        w0_l[pl.ds(k * _L, _L)] = zero16
        w1_l[pl.ds(k * _L, _L)] = zero16
    base = sid * _STRIPE
    for t in range(_STRIPE // _CHUNK):
        pltpu.sync_copy(rows0_l, num_s.at[pl.ds(base + t * _CHUNK, _CHUNK)])
        pltpu.sync_copy(w0_l, den_s.at[pl.ds(base + t * _CHUNK, _CHUNK)])
    plsc.subcore_barrier()

    def _compute(j, wbuf, rows):
        for k in range(_CHUNK // _L):
            si = src_l[j, pl.ds(k * _L, _L)]
            di = dst_l[j, pl.ds(k * _L, _L)]
            s = (plsc.load_gather(as_l, [si]) +
                 plsc.load_gather(ad_l, [di]))
            e = jnp.maximum(s, 0.2 * s)
            wbuf[pl.ds(k * _L, _L)] = jnp.exp(e - cbound)
        @plsc.parallel_loop(0, _CHUNK, unroll=8)
        def _scale(i):
            wv = plsc.load_gather(
                wbuf, [jnp.broadcast_to(i, (_L,)).astype(jnp.int32)])
            for f in range(half // _L):
                sl = pl.ds(f * _L, _L)
                rows[i, sl] = rows[i, sl] * wv

    def _issue_gather(j, rows, gsem):
        return pltpu.async_copy(hs_hbm.at[cid].at[src_l.at[j]], rows, gsem)

    def _issue_scatter(j, wbuf, rows, ssem):
        pltpu.async_copy(rows, num_s.at[dst_l.at[j]], ssem, add=True)
        pltpu.async_copy(wbuf, den_s.at[dst_l.at[j]], ssem, add=True)

    def _drain_gather(rows, gsem):
        pltpu.make_async_copy(hs_hbm.at[cid, pl.ds(0, _CHUNK)],
                              rows, gsem).wait()

    def _drain_scatter(wbuf, rows, ssem):
        pltpu.make_async_copy(hs_hbm.at[cid, pl.ds(0, _CHUNK)],
                              rows, ssem).wait()
        pltpu.make_async_copy(as_hbm.at[pl.ds(0, _CHUNK)],
                              wbuf, ssem).wait()

    _issue_scatter(0, w0_l, rows0_l, ssem0)
    _issue_scatter(0, w1_l, rows1_l, ssem1)
    _drain_scatter(w0_l, rows0_l, ssem0)
    _issue_gather(0, rows0_l, gsem0)

    def _pair(k, _):
        a = 2 * k
        b = a + 1
        _drain_gather(rows0_l, gsem0)
        _drain_scatter(w1_l, rows1_l, ssem1)
        _issue_gather(b, rows1_l, gsem1)
        _compute(a, w0_l, rows0_l)
        _issue_scatter(a, w0_l, rows0_l, ssem0)
        _drain_gather(rows1_l, gsem1)
        _compute(b, w1_l, rows1_l)
        _drain_scatter(w0_l, rows0_l, ssem0)
        _issue_gather(jnp.minimum(a + 2, _NCHUNK - 1), rows0_l, gsem0)
        _issue_scatter(b, w1_l, rows1_l, ssem1)
        return 0
    lax.fori_loop(0, _NCHUNK // 2, _pair, 0)
    _drain_gather(rows0_l, gsem0)
    _drain_scatter(w1_l, rows1_l, ssem1)
    plsc.subcore_barrier()

    pltpu.sync_copy(num_s.at[pl.ds(base, _STRIPE)],
                    num_out.at[cid, pl.ds(base, _STRIPE)])
    @pl.when(cid == 0)
    def _():
        pltpu.sync_copy(den_s.at[pl.ds(base, _STRIPE)],
                        den_out.at[pl.ds(base, _STRIPE)])


def _make_edge_kernel(Df):
    half = Df // 2
    mesh = plsc.VectorSubcoreMesh(core_axis_name="c", subcore_axis_name="s")
    return pl.kernel(
        functools.partial(_edge_kernel_body, Df),
        out_type=(jax.ShapeDtypeStruct((_NC, _NPAD, half), jnp.float32),
                  jax.ShapeDtypeStruct((_NPAD,), jnp.float32)),
        mesh=mesh,
        scratch_types=(
            pltpu.VMEM((_NPAD,), jnp.float32),
            pltpu.VMEM((_NPAD,), jnp.float32),
            pltpu.VMEM((_NCHUNK, _CHUNK), jnp.int32),
            pltpu.VMEM((_NCHUNK, _CHUNK), jnp.int32),
            pltpu.VMEM((_CHUNK,), jnp.float32),
            pltpu.VMEM((_CHUNK,), jnp.float32),
            pltpu.VMEM((_CHUNK, half), jnp.float32),
            pltpu.VMEM((_CHUNK, half), jnp.float32),
            pltpu.VMEM_SHARED((_NPAD, half), jnp.float32),
            pltpu.VMEM_SHARED((_NPAD,), jnp.float32),
            pltpu.SemaphoreType.DMA,
            pltpu.SemaphoreType.DMA,
            pltpu.SemaphoreType.DMA,
            pltpu.SemaphoreType.DMA,
        ),
        compiler_params=pltpu.CompilerParams(
            needs_layout_passes=False, use_tc_tiling_on_sc=False),
        name=f"gat_edge_sc_{Df}",
    )


def _tc_a_body(x_ref, w_ref, asr, adr, h_ref, sa_ref, da_ref):
    h = jnp.dot(x_ref[...], w_ref[...], preferred_element_type=jnp.float32)
    h_ref[...] = h
    sa_ref[...] = (h * asr[...]).sum(-1, keepdims=True)
    da_ref[...] = (h * adr[...]).sum(-1, keepdims=True)


def _tc_b_body(num_ref, den_ref, b1_ref, w2_ref, asr, adr,
               h2_ref, sa_ref, da_ref):
    d = den_ref[...] + 1e-30
    h = jax.nn.relu(num_ref[...] / d + b1_ref[...])
    h2 = jnp.dot(h, w2_ref[...], preferred_element_type=jnp.float32)
    h2_ref[...] = h2
    sa_ref[...] = (h2 * asr[...]).sum(-1, keepdims=True)
    da_ref[...] = (h2 * adr[...]).sum(-1, keepdims=True)


def _tc_c_body(num_ref, den_ref, b2_ref, boh_ref, lw_ref, lb_ref,
               res_ref, acc_ref):
    i = pl.program_id(0)
    @pl.when(i == 0)
    def _():
        acc_ref[...] = jnp.zeros_like(acc_ref)
    d = den_ref[...] + 1e-30
    o = num_ref[...] / d + b2_ref[...]
    z = o - o.max(-1, keepdims=True)
    ls = z - jnp.log(jnp.exp(z).sum(-1, keepdims=True))
    lsa = jnp.concatenate([ls, jnp.ones_like(ls)], axis=-1)
    acc_ref[...] += lax.dot_general(
        boh_ref[...], lsa, (((0,), (0,)), ((), ())),
        preferred_element_type=jnp.float32)
    @pl.when(i == pl.num_programs(0) - 1)
    def _():
        acc = acc_ref[...]
        pm = acc[:, :_OUT] / jnp.maximum(acc[:, _OUT:_OUT + 1], 1.0)
        res_ref[...] = jnp.dot(pm, lw_ref[...],
                               preferred_element_type=jnp.float32) + lb_ref[...]


_NB = _NPAD // _ROWBLK


def _tc_a(xp, W1, a1s, a1d):
    return pl.pallas_call(
        _tc_a_body,
        grid=(_NB,),
        in_specs=[
            pl.BlockSpec((_ROWBLK, _D), lambda i: (i, 0)),
            pl.BlockSpec((_D, _HID), lambda i: (0, 0)),
            pl.BlockSpec((1, _HID), lambda i: (0, 0)),
            pl.BlockSpec((1, _HID), lambda i: (0, 0)),
        ],
        out_specs=[
            pl.BlockSpec((_ROWBLK, _HID), lambda i: (i, 0)),
            pl.BlockSpec((_ROWBLK, 1), lambda i: (i, 0)),
            pl.BlockSpec((_ROWBLK, 1), lambda i: (i, 0)),
        ],
        out_shape=[
            jax.ShapeDtypeStruct((_NPAD, _HID), jnp.float32),
            jax.ShapeDtypeStruct((_NPAD, 1), jnp.float32),
            jax.ShapeDtypeStruct((_NPAD, 1), jnp.float32),
        ],
    )(xp, W1, a1s, a1d)


def _tc_b(num1, den1, b1, W2, a2s, a2d):
    return pl.pallas_call(
        _tc_b_body,
        grid=(_NB,),
        in_specs=[
            pl.BlockSpec((_ROWBLK, _HID), lambda i: (i, 0)),
            pl.BlockSpec((_ROWBLK, 1), lambda i: (i, 0)),
            pl.BlockSpec((1, _HID), lambda i: (0, 0)),
            pl.BlockSpec((_HID, _OUT), lambda i: (0, 0)),
            pl.BlockSpec((1, _OUT), lambda i: (0, 0)),
            pl.BlockSpec((1, _OUT), lambda i: (0, 0)),
        ],
        out_specs=[
            pl.BlockSpec((_ROWBLK, _OUT), lambda i: (i, 0)),
            pl.BlockSpec((_ROWBLK, 1), lambda i: (i, 0)),
            pl.BlockSpec((_ROWBLK, 1), lambda i: (i, 0)),
        ],
        out_shape=[
            jax.ShapeDtypeStruct((_NPAD, _OUT), jnp.float32),
            jax.ShapeDtypeStruct((_NPAD, 1), jnp.float32),
            jax.ShapeDtypeStruct((_NPAD, 1), jnp.float32),
        ],
    )(num1, den1, b1, W2, a2s, a2d)


def _tc_c(num2, den2, b2, boh, lwp, lbp):
    return pl.pallas_call(
        _tc_c_body,
        grid=(_NB,),
        in_specs=[
            pl.BlockSpec((_ROWBLK, _OUT), lambda i: (i, 0)),
            pl.BlockSpec((_ROWBLK, 1), lambda i: (i, 0)),
            pl.BlockSpec((1, _OUT), lambda i: (0, 0)),
            pl.BlockSpec((_ROWBLK, _G), lambda i: (i, 0)),
            pl.BlockSpec((_OUT, _G), lambda i: (0, 0)),
            pl.BlockSpec((1, _G), lambda i: (0, 0)),
        ],
        out_specs=pl.BlockSpec((_G, _G), lambda i: (0, 0)),
        out_shape=jax.ShapeDtypeStruct((_G, _G), jnp.float32),
        scratch_shapes=[pltpu.VMEM((_G, _G), jnp.float32)],
    )(num2, den2, b2, boh, lwp, lbp)


def _split_cols(h, Df):
    half = Df // 2
    return jnp.stack([h[:, :half], h[:, half:]], axis=0)


def kernel(x, adj, batch, W1, a1_src, a1_dst, b1, W2, a2_src, a2_dst, b2,
           linW, linb):
    xp = jnp.pad(x, ((0, _NPAD - _N), (0, 0)))
    loops = jnp.arange(_N, dtype=jnp.int32)
    src_all = jnp.concatenate(
        [adj[0], loops, jnp.zeros((_EPAD - _ETOT,), jnp.int32)])
    dst_all = jnp.concatenate(
        [adj[1], loops, jnp.full((_EPAD - _ETOT,), _N, jnp.int32)])
    srcg = src_all.reshape(_NS, _NCHUNK, _CHUNK)
    dstg = dst_all.reshape(_NS, _NCHUNK, _CHUNK)
    batch_pad = jnp.concatenate(
        [batch, jnp.full((_NPAD - _N,), _G, jnp.int32)])
    boh = jax.nn.one_hot(batch_pad, _G, dtype=jnp.float32)
    lwp = jnp.pad(linW, ((0, 0), (0, _G - 1)))
    lbp = jnp.broadcast_to(linb.reshape(1, 1), (1, _G))
    b1r = b1.reshape(1, _HID)
    b2r = b2.reshape(1, _OUT)

    h1, as1, ad1 = _tc_a(xp, W1, a1_src, a1_dst)
    ek1 = _make_edge_kernel(_HID)
    num1p, den1 = ek1(_split_cols(h1, _HID), as1.reshape(_NPAD),
                      ad1.reshape(_NPAD), srcg, dstg)
    num1 = jnp.concatenate([num1p[0], num1p[1]], axis=1)

    h2, as2, ad2 = _tc_b(num1, den1.reshape(_NPAD, 1), b1r, W2,
                         a2_src, a2_dst)
    ek2 = _make_edge_kernel(_OUT)
    num2p, den2 = ek2(_split_cols(h2, _OUT), as2.reshape(_NPAD),
                      ad2.reshape(_NPAD), srcg, dstg)
    num2 = jnp.concatenate([num2p[0], num2p[1]], axis=1)

    res = _tc_c(num2, den2.reshape(_NPAD, 1), b2r, boh, lwp, lbp)
    return res[:, :1]


# speedup vs baseline: 41.6196x; 1.2155x over previous
import functools

import jax
import jax.numpy as jnp
from jax import lax
from jax.experimental import pallas as pl
from jax.experimental.pallas import tpu as pltpu
from jax.experimental.pallas import tpu_sc as plsc

_N = 10000
_E = 320000
_D = 128
_HID = 128
_OUT = 64
_G = 128

_NC, _NS, _L = 2, 16, 16
_CHUNK = 128
_ETOT = _E + _N
_NCHUNK = -(-_ETOT // (_NS * _CHUNK))
_EPAD = _NS * _NCHUNK * _CHUNK
_NPAD = 10240
_STRIPE = _NPAD // _NS
_ROWBLK = 1024


def _edge_kernel_body(Df, hs_hbm, as_hbm, ad_hbm, src_hbm, dst_hbm,
                      num_out, den_out,
                      as_l, ad_l, src_l, dst_l,
                      w0_l, w1_l, rows0_l, rows1_l,
                      num_s, den_s, gsem0, gsem1, ssem0, ssem1):
    half = Df // 2
    cid = lax.axis_index("c")
    sid = lax.axis_index("s")
    w_l = w0_l

    pltpu.sync_copy(as_hbm, as_l)
    pltpu.sync_copy(ad_hbm, ad_l)
    pltpu.sync_copy(src_hbm.at[sid], src_l)
    pltpu.sync_copy(dst_hbm.at[sid], dst_l)

    def _maxstep_a(i, m):
        return jnp.maximum(m, as_l[pl.ds(i * _L, _L)])
    def _maxstep_b(i, m):
        return jnp.maximum(m, ad_l[pl.ds(i * _L, _L)])
    neg = jnp.full((_L,), -3e38, jnp.float32)

    def _lane_max_splat(v16):
        w_l[pl.ds(0, _L)] = v16
        acc = plsc.load_gather(w_l, [jnp.zeros((_L,), jnp.int32)])
        for k in range(1, _L):
            acc = jnp.maximum(
                acc, plsc.load_gather(w_l, [jnp.full((_L,), k, jnp.int32)]))
        return acc

    masv = _lane_max_splat(lax.fori_loop(0, _NPAD // _L, _maxstep_a, neg))
    madv = _lane_max_splat(lax.fori_loop(0, _NPAD // _L, _maxstep_b, neg))
    msum = masv + madv
    cbound = jnp.maximum(msum, 0.2 * msum)

    zero16 = jnp.zeros((_L,), jnp.float32)
    def _zrow(i, _):
        for f in range(half // _L):
            rows0_l[i, pl.ds(f * _L, _L)] = zero16
            rows1_l[i, pl.ds(f * _L, _L)] = zero16
        return 0
    lax.fori_loop(0, _CHUNK, _zrow, 0)
    for k in range(_CHUNK // _L):
        w0_l[pl.ds(k * _L, _L)] = zero16
        w1_l[pl.ds(k * _L, _L)] = zero16
    base = sid * _STRIPE
    for t in range(_STRIPE // _CHUNK):
        pltpu.sync_copy(rows0_l, num_s.at[pl.ds(base + t * _CHUNK, _CHUNK)])
        pltpu.sync_copy(w0_l, den_s.at[pl.ds(base + t * _CHUNK, _CHUNK)])
    plsc.subcore_barrier()

    def _weights(j, wbuf):
        @plsc.parallel_loop(0, _CHUNK // _L, unroll=4)
        def _wgrp(k):
            si = src_l[j, pl.ds(k * _L, _L)]
            di = dst_l[j, pl.ds(k * _L, _L)]
            s = (plsc.load_gather(as_l, [si]) +
                 plsc.load_gather(ad_l, [di]))
            e = jnp.maximum(s, 0.2 * s)
            wbuf[pl.ds(k * _L, _L)] = jnp.exp(e - cbound)

    def _scale_rows(wbuf, rows):
        @plsc.parallel_loop(0, _CHUNK, unroll=8)
        def _scale(i):
            wv = plsc.load_gather(
                wbuf, [jnp.broadcast_to(i, (_L,)).astype(jnp.int32)])
            for f in range(half // _L):
                sl = pl.ds(f * _L, _L)
                rows[i, sl] = rows[i, sl] * wv

    def _issue_gather(j, rows, gsem):
        return pltpu.async_copy(hs_hbm.at[cid].at[src_l.at[j]], rows, gsem)

    def _issue_scatter(j, wbuf, rows, ssem):
        pltpu.async_copy(rows, num_s.at[dst_l.at[j]], ssem, add=True)
        pltpu.async_copy(wbuf, den_s.at[dst_l.at[j]], ssem, add=True)

    def _drain_gather(rows, gsem):
        pltpu.make_async_copy(hs_hbm.at[cid, pl.ds(0, _CHUNK)],
                              rows, gsem).wait()

    def _drain_scatter(wbuf, rows, ssem):
        pltpu.make_async_copy(hs_hbm.at[cid, pl.ds(0, _CHUNK)],
                              rows, ssem).wait()
        pltpu.make_async_copy(as_hbm.at[pl.ds(0, _CHUNK)],
                              wbuf, ssem).wait()

    _issue_scatter(0, w0_l, rows0_l, ssem0)
    _issue_scatter(0, w1_l, rows1_l, ssem1)
    _drain_scatter(w0_l, rows0_l, ssem0)
    _issue_gather(0, rows0_l, gsem0)

    def _pair(k, _):
        a = 2 * k
        b = a + 1
        _drain_scatter(w1_l, rows1_l, ssem1)
        _issue_gather(b, rows1_l, gsem1)
        _weights(a, w0_l)
        _drain_gather(rows0_l, gsem0)
        _scale_rows(w0_l, rows0_l)
        _issue_scatter(a, w0_l, rows0_l, ssem0)
        _weights(b, w1_l)
        _drain_gather(rows1_l, gsem1)
        _scale_rows(w1_l, rows1_l)
        _drain_scatter(w0_l, rows0_l, ssem0)
        _issue_gather(jnp.minimum(a + 2, _NCHUNK - 1), rows0_l, gsem0)
        _issue_scatter(b, w1_l, rows1_l, ssem1)
        return 0
    lax.fori_loop(0, _NCHUNK // 2, _pair, 0)
    _drain_gather(rows0_l, gsem0)
    _drain_scatter(w1_l, rows1_l, ssem1)
    plsc.subcore_barrier()

    pltpu.sync_copy(num_s.at[pl.ds(base, _STRIPE)],
                    num_out.at[cid, pl.ds(base, _STRIPE)])
    @pl.when(cid == 0)
    def _():
        pltpu.sync_copy(den_s.at[pl.ds(base, _STRIPE)],
                        den_out.at[pl.ds(base, _STRIPE)])


def _make_edge_kernel(Df):
    half = Df // 2
    mesh = plsc.VectorSubcoreMesh(core_axis_name="c", subcore_axis_name="s")
    return pl.kernel(
        functools.partial(_edge_kernel_body, Df),
        out_type=(jax.ShapeDtypeStruct((_NC, _NPAD, half), jnp.float32),
                  jax.ShapeDtypeStruct((_NPAD,), jnp.float32)),
        mesh=mesh,
        scratch_types=(
            pltpu.VMEM((_NPAD,), jnp.float32),
            pltpu.VMEM((_NPAD,), jnp.float32),
            pltpu.VMEM((_NCHUNK, _CHUNK), jnp.int32),
            pltpu.VMEM((_NCHUNK, _CHUNK), jnp.int32),
            pltpu.VMEM((_CHUNK,), jnp.float32),
            pltpu.VMEM((_CHUNK,), jnp.float32),
            pltpu.VMEM((_CHUNK, half), jnp.float32),
            pltpu.VMEM((_CHUNK, half), jnp.float32),
            pltpu.VMEM_SHARED((_NPAD, half), jnp.float32),
            pltpu.VMEM_SHARED((_NPAD,), jnp.float32),
            pltpu.SemaphoreType.DMA,
            pltpu.SemaphoreType.DMA,
            pltpu.SemaphoreType.DMA,
            pltpu.SemaphoreType.DMA,
        ),
        compiler_params=pltpu.CompilerParams(
            needs_layout_passes=False, use_tc_tiling_on_sc=False),
        name=f"gat_edge_sc_{Df}",
    )


def _tc_a_body(x_ref, w_ref, asr, adr, h_ref, sa_ref, da_ref):
    h = jnp.dot(x_ref[...], w_ref[...], preferred_element_type=jnp.float32)
    h_ref[...] = h
    sa_ref[...] = (h * asr[...]).sum(-1, keepdims=True)
    da_ref[...] = (h * adr[...]).sum(-1, keepdims=True)


def _tc_b_body(num_ref, den_ref, b1_ref, w2_ref, asr, adr,
               h2_ref, sa_ref, da_ref):
    d = den_ref[...] + 1e-30
    nsum = jnp.concatenate([num_ref[0], num_ref[1]], axis=-1)
    h = jax.nn.relu(nsum / d + b1_ref[...])
    h2 = jnp.dot(h, w2_ref[...], preferred_element_type=jnp.float32)
    h2_ref[...] = h2
    sa_ref[...] = (h2 * asr[...]).sum(-1, keepdims=True)
    da_ref[...] = (h2 * adr[...]).sum(-1, keepdims=True)


def _tc_c_body(num_ref, den_ref, b2_ref, bat_ref, lw_ref, lb_ref,
               res_ref, acc_ref):
    i = pl.program_id(0)
    @pl.when(i == 0)
    def _():
        acc_ref[...] = jnp.zeros_like(acc_ref)
    d = den_ref[...] + 1e-30
    nsum = jnp.concatenate([num_ref[0], num_ref[1]], axis=-1)
    o = nsum / d + b2_ref[...]
    z = o - o.max(-1, keepdims=True)
    ls = z - jnp.log(jnp.exp(z).sum(-1, keepdims=True))
    lsa = jnp.concatenate([ls, jnp.ones_like(ls)], axis=-1)
    boh = (bat_ref[...] == lax.broadcasted_iota(
        jnp.int32, (1, _G), 1)).astype(jnp.float32)
    acc_ref[...] += lax.dot_general(
        boh, lsa, (((0,), (0,)), ((), ())),
        preferred_element_type=jnp.float32)
    @pl.when(i == pl.num_programs(0) - 1)
    def _():
        acc = acc_ref[...]
        pm = acc[:, :_OUT] / jnp.maximum(acc[:, _OUT:_OUT + 1], 1.0)
        res_ref[...] = jnp.dot(pm, lw_ref[...],
                               preferred_element_type=jnp.float32) + lb_ref[...]


_NB = _NPAD // _ROWBLK


def _tc_a(xp, W1, a1s, a1d):
    return pl.pallas_call(
        _tc_a_body,
        grid=(_NB,),
        in_specs=[
            pl.BlockSpec((_ROWBLK, _D), lambda i: (i, 0)),
            pl.BlockSpec((_D, _HID), lambda i: (0, 0)),
            pl.BlockSpec((1, _HID), lambda i: (0, 0)),
            pl.BlockSpec((1, _HID), lambda i: (0, 0)),
        ],
        out_specs=[
            pl.BlockSpec((_ROWBLK, _HID), lambda i: (i, 0)),
            pl.BlockSpec((_ROWBLK, 1), lambda i: (i, 0)),
            pl.BlockSpec((_ROWBLK, 1), lambda i: (i, 0)),
        ],
        out_shape=[
            jax.ShapeDtypeStruct((_NPAD, _HID), jnp.float32),
            jax.ShapeDtypeStruct((_NPAD, 1), jnp.float32),
            jax.ShapeDtypeStruct((_NPAD, 1), jnp.float32),
        ],
    )(xp, W1, a1s, a1d)


def _tc_b(num1, den1, b1, W2, a2s, a2d):
    return pl.pallas_call(
        _tc_b_body,
        grid=(_NB,),
        in_specs=[
            pl.BlockSpec((_NC, _ROWBLK, _HID // 2), lambda i: (0, i, 0)),
            pl.BlockSpec((_ROWBLK, 1), lambda i: (i, 0)),
            pl.BlockSpec((1, _HID), lambda i: (0, 0)),
            pl.BlockSpec((_HID, _OUT), lambda i: (0, 0)),
            pl.BlockSpec((1, _OUT), lambda i: (0, 0)),
            pl.BlockSpec((1, _OUT), lambda i: (0, 0)),
        ],
        out_specs=[
            pl.BlockSpec((_ROWBLK, _OUT), lambda i: (i, 0)),
            pl.BlockSpec((_ROWBLK, 1), lambda i: (i, 0)),
            pl.BlockSpec((_ROWBLK, 1), lambda i: (i, 0)),
        ],
        out_shape=[
            jax.ShapeDtypeStruct((_NPAD, _OUT), jnp.float32),
            jax.ShapeDtypeStruct((_NPAD, 1), jnp.float32),
            jax.ShapeDtypeStruct((_NPAD, 1), jnp.float32),
        ],
    )(num1, den1, b1, W2, a2s, a2d)


def _tc_c(num2, den2, b2, bat2d, lwp, lbp):
    return pl.pallas_call(
        _tc_c_body,
        grid=(_NB,),
        in_specs=[
            pl.BlockSpec((_NC, _ROWBLK, _OUT // 2), lambda i: (0, i, 0)),
            pl.BlockSpec((_ROWBLK, 1), lambda i: (i, 0)),
            pl.BlockSpec((1, _OUT), lambda i: (0, 0)),
            pl.BlockSpec((_ROWBLK, 1), lambda i: (i, 0)),
            pl.BlockSpec((_OUT, _G), lambda i: (0, 0)),
            pl.BlockSpec((1, _G), lambda i: (0, 0)),
        ],
        out_specs=pl.BlockSpec((_G, _G), lambda i: (0, 0)),
        out_shape=jax.ShapeDtypeStruct((_G, _G), jnp.float32),
        scratch_shapes=[pltpu.VMEM((_G, _G), jnp.float32)],
    )(num2, den2, b2, bat2d, lwp, lbp)


def _split_cols(h, Df):
    half = Df // 2
    return jnp.stack([h[:, :half], h[:, half:]], axis=0)


def kernel(x, adj, batch, W1, a1_src, a1_dst, b1, W2, a2_src, a2_dst, b2,
           linW, linb):
    xp = jnp.pad(x, ((0, _NPAD - _N), (0, 0)))
    loops = jnp.arange(_N, dtype=jnp.int32)
    src_all = jnp.concatenate(
        [adj[0], loops, jnp.zeros((_EPAD - _ETOT,), jnp.int32)])
    dst_all = jnp.concatenate(
        [adj[1], loops, jnp.full((_EPAD - _ETOT,), _N, jnp.int32)])
    srcg = src_all.reshape(_NS, _NCHUNK, _CHUNK)
    dstg = dst_all.reshape(_NS, _NCHUNK, _CHUNK)
    batch_pad = jnp.concatenate(
        [batch, jnp.full((_NPAD - _N,), _G, jnp.int32)])
    bat2d = batch_pad.reshape(_NPAD, 1)
    lwp = jnp.pad(linW, ((0, 0), (0, _G - 1)))
    lbp = jnp.broadcast_to(linb.reshape(1, 1), (1, _G))
    b1r = b1.reshape(1, _HID)
    b2r = b2.reshape(1, _OUT)

    h1, as1, ad1 = _tc_a(xp, W1, a1_src, a1_dst)
    ek1 = _make_edge_kernel(_HID)
    num1p, den1 = ek1(_split_cols(h1, _HID), as1.reshape(_NPAD),
                      ad1.reshape(_NPAD), srcg, dstg)

    h2, as2, ad2 = _tc_b(num1p, den1.reshape(_NPAD, 1), b1r, W2,
                         a2_src, a2_dst)
    ek2 = _make_edge_kernel(_OUT)
    num2p, den2 = ek2(_split_cols(h2, _OUT), as2.reshape(_NPAD),
                      ad2.reshape(_NPAD), srcg, dstg)

    res = _tc_c(num2p, den2.reshape(_NPAD, 1), b2r, bat2d, lwp, lbp)
    return res[:, :1]


# speedup vs baseline: 44.5448x; 1.0703x over previous
import functools

import jax
import jax.numpy as jnp
from jax import lax
from jax.experimental import pallas as pl
from jax.experimental.pallas import tpu as pltpu
from jax.experimental.pallas import tpu_sc as plsc

_N = 10000
_E = 320000
_D = 128
_HID = 128
_OUT = 64
_G = 128

_NC, _NS, _L = 2, 16, 16
_CHUNK = 128
_ETOT = _E + _N
_NCHUNK = -(-_ETOT // (_NS * _CHUNK))
_EPAD = _NS * _NCHUNK * _CHUNK
_NPAD = 10240
_STRIPE = _NPAD // _NS
_ROWBLK = 1024


def _edge_kernel_body(Df, hs_hbm, as_hbm, ad_hbm, src_hbm, dst_hbm,
                      num_out, den_out,
                      as_l, ad_l, src_l, dst_l,
                      w0_l, w1_l, rows0_l, rows1_l,
                      num_s, den_s, gsem0, gsem1, ssem0, ssem1):
    half = Df // 2
    cid = lax.axis_index("c")
    sid = lax.axis_index("s")
    w_l = w0_l

    pltpu.sync_copy(as_hbm, as_l)
    pltpu.sync_copy(ad_hbm, ad_l)
    pltpu.sync_copy(src_hbm.at[sid], src_l)
    pltpu.sync_copy(dst_hbm.at[sid], dst_l)

    def _maxstep_a(i, m):
        return jnp.maximum(m, as_l[pl.ds(i * _L, _L)])
    def _maxstep_b(i, m):
        return jnp.maximum(m, ad_l[pl.ds(i * _L, _L)])
    neg = jnp.full((_L,), -3e38, jnp.float32)

    def _lane_max_splat(v16):
        w_l[pl.ds(0, _L)] = v16
        acc = plsc.load_gather(w_l, [jnp.zeros((_L,), jnp.int32)])
        for k in range(1, _L):
            acc = jnp.maximum(
                acc, plsc.load_gather(w_l, [jnp.full((_L,), k, jnp.int32)]))
        return acc

    masv = _lane_max_splat(lax.fori_loop(0, _NPAD // _L, _maxstep_a, neg))
    madv = _lane_max_splat(lax.fori_loop(0, _NPAD // _L, _maxstep_b, neg))
    msum = masv + madv
    cbound = jnp.maximum(msum, 0.2 * msum)

    zero16 = jnp.zeros((_L,), jnp.float32)
    def _zrow(i, _):
        for f in range(half // _L):
            rows0_l[i, pl.ds(f * _L, _L)] = zero16
            rows1_l[i, pl.ds(f * _L, _L)] = zero16
        return 0
    lax.fori_loop(0, _CHUNK, _zrow, 0)
    for k in range(_CHUNK // _L):
        w0_l[pl.ds(k * _L, _L)] = zero16
        w1_l[pl.ds(k * _L, _L)] = zero16
    base = sid * _STRIPE
    for t in range(_STRIPE // _CHUNK):
        pltpu.sync_copy(rows0_l, num_s.at[pl.ds(base + t * _CHUNK, _CHUNK)])
        pltpu.sync_copy(w0_l, den_s.at[pl.ds(base + t * _CHUNK, _CHUNK)])
    plsc.subcore_barrier()

    def _weights(j, wbuf):
        @plsc.parallel_loop(0, _CHUNK // _L, unroll=4)
        def _wgrp(k):
            si = src_l[j, pl.ds(k * _L, _L)]
            di = dst_l[j, pl.ds(k * _L, _L)]
            s = (plsc.load_gather(as_l, [si]) +
                 plsc.load_gather(ad_l, [di]))
            e = jnp.maximum(s, 0.2 * s)
            wbuf[pl.ds(k * _L, _L)] = jnp.exp(e - cbound)

    def _scale_rows(wbuf, rows):
        @plsc.parallel_loop(0, _CHUNK, unroll=8)
        def _scale(i):
            wv = plsc.load_gather(
                wbuf, [jnp.broadcast_to(i, (_L,)).astype(jnp.int32)])
            for f in range(half // _L):
                sl = pl.ds(f * _L, _L)
                rows[i, sl] = rows[i, sl] * wv

    def _issue_gather(j, rows, gsem):
        return pltpu.async_copy(hs_hbm.at[cid].at[src_l.at[j]], rows, gsem)

    def _issue_scatter(j, wbuf, rows, ssem):
        pltpu.async_copy(rows, num_s.at[dst_l.at[j]], ssem, add=True)
        pltpu.async_copy(wbuf, den_s.at[dst_l.at[j]], ssem, add=True)

    def _drain_gather(rows, gsem):
        pltpu.make_async_copy(hs_hbm.at[cid, pl.ds(0, _CHUNK)],
                              rows, gsem).wait()

    def _drain_scatter(wbuf, rows, ssem):
        pltpu.make_async_copy(hs_hbm.at[cid, pl.ds(0, _CHUNK)],
                              rows, ssem).wait()
        pltpu.make_async_copy(as_hbm.at[pl.ds(0, _CHUNK)],
                              wbuf, ssem).wait()

    _issue_scatter(0, w0_l, rows0_l, ssem0)
    _issue_scatter(0, w1_l, rows1_l, ssem1)
    _drain_scatter(w0_l, rows0_l, ssem0)
    _issue_gather(0, rows0_l, gsem0)

    def _pair(k, _):
        a = 2 * k
        b = a + 1
        _drain_scatter(w1_l, rows1_l, ssem1)
        _issue_gather(b, rows1_l, gsem1)
        _weights(a, w0_l)
        _drain_gather(rows0_l, gsem0)
        _scale_rows(w0_l, rows0_l)
        _issue_scatter(a, w0_l, rows0_l, ssem0)
        _weights(b, w1_l)
        _drain_gather(rows1_l, gsem1)
        _scale_rows(w1_l, rows1_l)
        _drain_scatter(w0_l, rows0_l, ssem0)
        _issue_gather(jnp.minimum(a + 2, _NCHUNK - 1), rows0_l, gsem0)
        _issue_scatter(b, w1_l, rows1_l, ssem1)
        return 0
    lax.fori_loop(0, _NCHUNK // 2, _pair, 0)
    _drain_gather(rows0_l, gsem0)
    _drain_scatter(w1_l, rows1_l, ssem1)
    plsc.subcore_barrier()

    pltpu.sync_copy(num_s.at[pl.ds(base, _STRIPE)],
                    num_out.at[cid, pl.ds(base, _STRIPE)])
    @pl.when(cid == 0)
    def _():
        pltpu.sync_copy(den_s.at[pl.ds(base, _STRIPE)],
                        den_out.at[pl.ds(base, _STRIPE)])


def _make_edge_kernel(Df):
    half = Df // 2
    mesh = plsc.VectorSubcoreMesh(core_axis_name="c", subcore_axis_name="s")
    return pl.kernel(
        functools.partial(_edge_kernel_body, Df),
        out_type=(jax.ShapeDtypeStruct((_NC, _NPAD, half), jnp.float32),
                  jax.ShapeDtypeStruct((_NPAD,), jnp.float32)),
        mesh=mesh,
        scratch_types=(
            pltpu.VMEM((_NPAD,), jnp.float32),
            pltpu.VMEM((_NPAD,), jnp.float32),
            pltpu.VMEM((_NCHUNK, _CHUNK), jnp.int32),
            pltpu.VMEM((_NCHUNK, _CHUNK), jnp.int32),
            pltpu.VMEM((_CHUNK,), jnp.float32),
            pltpu.VMEM((_CHUNK,), jnp.float32),
            pltpu.VMEM((_CHUNK, half), jnp.float32),
            pltpu.VMEM((_CHUNK, half), jnp.float32),
            pltpu.VMEM_SHARED((_NPAD, half), jnp.float32),
            pltpu.VMEM_SHARED((_NPAD,), jnp.float32),
            pltpu.SemaphoreType.DMA,
            pltpu.SemaphoreType.DMA,
            pltpu.SemaphoreType.DMA,
            pltpu.SemaphoreType.DMA,
        ),
        compiler_params=pltpu.CompilerParams(
            needs_layout_passes=False, use_tc_tiling_on_sc=False),
        name=f"gat_edge_sc_{Df}",
    )


def _tc_a_body(x_ref, w_ref, asr, adr, h_ref, sa_ref, da_ref):
    h = jnp.dot(x_ref[...], w_ref[...], preferred_element_type=jnp.float32)
    h_ref[0] = h[:, :_HID // 2]
    h_ref[1] = h[:, _HID // 2:]
    sa_ref[...] = (h * asr[...]).sum(-1, keepdims=True).reshape(8, _G)
    da_ref[...] = (h * adr[...]).sum(-1, keepdims=True).reshape(8, _G)


def _tc_b_body(num_ref, den_ref, b1_ref, w2_ref, asr, adr,
               h2_ref, sa_ref, da_ref):
    d = den_ref[...] + 1e-30
    nsum = jnp.concatenate([num_ref[0], num_ref[1]], axis=-1)
    h = jax.nn.relu(nsum / d + b1_ref[...])
    h2 = jnp.dot(h, w2_ref[...], preferred_element_type=jnp.float32)
    h2_ref[0] = h2[:, :_OUT // 2]
    h2_ref[1] = h2[:, _OUT // 2:]
    sa_ref[...] = (h2 * asr[...]).sum(-1, keepdims=True).reshape(8, _G)
    da_ref[...] = (h2 * adr[...]).sum(-1, keepdims=True).reshape(8, _G)


def _tc_c_body(num_ref, den_ref, b2_ref, bat_ref, lw_ref, lb_ref,
               res_ref, acc_ref):
    i = pl.program_id(0)
    @pl.when(i == 0)
    def _():
        acc_ref[...] = jnp.zeros_like(acc_ref)
    d = den_ref[...] + 1e-30
    nsum = jnp.concatenate([num_ref[0], num_ref[1]], axis=-1)
    o = nsum / d + b2_ref[...]
    z = o - o.max(-1, keepdims=True)
    ls = z - jnp.log(jnp.exp(z).sum(-1, keepdims=True))
    lsa = jnp.concatenate([ls, jnp.ones_like(ls)], axis=-1)
    boh = (bat_ref[...] == lax.broadcasted_iota(
        jnp.int32, (1, _G), 1)).astype(jnp.float32)
    acc_ref[...] += lax.dot_general(
        boh, lsa, (((0,), (0,)), ((), ())),
        preferred_element_type=jnp.float32)
    @pl.when(i == pl.num_programs(0) - 1)
    def _():
        acc = acc_ref[...]
        pm = acc[:, :_OUT] / jnp.maximum(acc[:, _OUT:_OUT + 1], 1.0)
        res_ref[...] = jnp.dot(pm, lw_ref[...],
                               preferred_element_type=jnp.float32) + lb_ref[...]


_NB = _NPAD // _ROWBLK


def _tc_a(xp, W1, a1s, a1d):
    return pl.pallas_call(
        _tc_a_body,
        grid=(_NB,),
        in_specs=[
            pl.BlockSpec((_ROWBLK, _D), lambda i: (i, 0)),
            pl.BlockSpec((_D, _HID), lambda i: (0, 0)),
            pl.BlockSpec((1, _HID), lambda i: (0, 0)),
            pl.BlockSpec((1, _HID), lambda i: (0, 0)),
        ],
        out_specs=[
            pl.BlockSpec((_NC, _ROWBLK, _HID // 2), lambda i: (0, i, 0)),
            pl.BlockSpec((8, _G), lambda i: (i, 0)),
            pl.BlockSpec((8, _G), lambda i: (i, 0)),
        ],
        out_shape=[
            jax.ShapeDtypeStruct((_NC, _NPAD, _HID // 2), jnp.float32),
            jax.ShapeDtypeStruct((_NB * 8, _G), jnp.float32),
            jax.ShapeDtypeStruct((_NB * 8, _G), jnp.float32),
        ],
    )(xp, W1, a1s, a1d)


def _tc_b(num1, den1, b1, W2, a2s, a2d):
    return pl.pallas_call(
        _tc_b_body,
        grid=(_NB,),
        in_specs=[
            pl.BlockSpec((_NC, _ROWBLK, _HID // 2), lambda i: (0, i, 0)),
            pl.BlockSpec((_ROWBLK, 1), lambda i: (i, 0)),
            pl.BlockSpec((1, _HID), lambda i: (0, 0)),
            pl.BlockSpec((_HID, _OUT), lambda i: (0, 0)),
            pl.BlockSpec((1, _OUT), lambda i: (0, 0)),
            pl.BlockSpec((1, _OUT), lambda i: (0, 0)),
        ],
        out_specs=[
            pl.BlockSpec((_NC, _ROWBLK, _OUT // 2), lambda i: (0, i, 0)),
            pl.BlockSpec((8, _G), lambda i: (i, 0)),
            pl.BlockSpec((8, _G), lambda i: (i, 0)),
        ],
        out_shape=[
            jax.ShapeDtypeStruct((_NC, _NPAD, _OUT // 2), jnp.float32),
            jax.ShapeDtypeStruct((_NB * 8, _G), jnp.float32),
            jax.ShapeDtypeStruct((_NB * 8, _G), jnp.float32),
        ],
    )(num1, den1, b1, W2, a2s, a2d)


def _tc_c(num2, den2, b2, bat2d, lwp, lbp):
    return pl.pallas_call(
        _tc_c_body,
        grid=(_NB,),
        in_specs=[
            pl.BlockSpec((_NC, _ROWBLK, _OUT // 2), lambda i: (0, i, 0)),
            pl.BlockSpec((_ROWBLK, 1), lambda i: (i, 0)),
            pl.BlockSpec((1, _OUT), lambda i: (0, 0)),
            pl.BlockSpec((_ROWBLK, 1), lambda i: (i, 0)),
            pl.BlockSpec((_OUT, _G), lambda i: (0, 0)),
            pl.BlockSpec((1, _G), lambda i: (0, 0)),
        ],
        out_specs=pl.BlockSpec((_G, _G), lambda i: (0, 0)),
        out_shape=jax.ShapeDtypeStruct((_G, _G), jnp.float32),
        scratch_shapes=[pltpu.VMEM((_G, _G), jnp.float32)],
    )(num2, den2, b2, bat2d, lwp, lbp)


def kernel(x, adj, batch, W1, a1_src, a1_dst, b1, W2, a2_src, a2_dst, b2,
           linW, linb):
    xp = jnp.pad(x, ((0, _NPAD - _N), (0, 0)))
    loops = jnp.arange(_N, dtype=jnp.int32)
    src_all = jnp.concatenate(
        [adj[0], loops, jnp.zeros((_EPAD - _ETOT,), jnp.int32)])
    dst_all = jnp.concatenate(
        [adj[1], loops, jnp.full((_EPAD - _ETOT,), _N, jnp.int32)])
    srcg = src_all.reshape(_NS, _NCHUNK, _CHUNK)
    dstg = dst_all.reshape(_NS, _NCHUNK, _CHUNK)
    batch_pad = jnp.concatenate(
        [batch, jnp.full((_NPAD - _N,), _G, jnp.int32)])
    bat2d = batch_pad.reshape(_NPAD, 1)
    lwp = jnp.pad(linW, ((0, 0), (0, _G - 1)))
    lbp = jnp.broadcast_to(linb.reshape(1, 1), (1, _G))
    b1r = b1.reshape(1, _HID)
    b2r = b2.reshape(1, _OUT)

    hs1, as1, ad1 = _tc_a(xp, W1, a1_src, a1_dst)
    ek1 = _make_edge_kernel(_HID)
    num1p, den1 = ek1(hs1, as1.reshape(_NPAD), ad1.reshape(_NPAD),
                      srcg, dstg)

    hs2, as2, ad2 = _tc_b(num1p, den1.reshape(_NPAD, 1), b1r, W2,
                          a2_src, a2_dst)
    ek2 = _make_edge_kernel(_OUT)
    num2p, den2 = ek2(hs2, as2.reshape(_NPAD), ad2.reshape(_NPAD),
                      srcg, dstg)

    res = _tc_c(num2p, den2.reshape(_NPAD, 1), b2r, bat2d, lwp, lbp)
    return res[:, :1]


# speedup vs baseline: 44.6514x; 1.0024x over previous
import functools

import jax
import jax.numpy as jnp
from jax import lax
from jax.experimental import pallas as pl
from jax.experimental.pallas import tpu as pltpu
from jax.experimental.pallas import tpu_sc as plsc

_N = 10000
_E = 320000
_D = 128
_HID = 128
_OUT = 64
_G = 128

_NC, _NS, _L = 2, 16, 16
_CHUNK = 128
_ETOT = _E + _N
_NCHUNK = -(-_ETOT // (_NS * _CHUNK))
_EPAD = _NS * _NCHUNK * _CHUNK
_NPAD = 10240
_STRIPE = _NPAD // _NS
_ROWBLK = 1024


def _edge_kernel_body(Df, hs_hbm, as_hbm, ad_hbm, src_hbm, dst_hbm,
                      num_out, den_out,
                      as_l, ad_l, src_l, dst_l,
                      w0_l, w1_l, rows0_l, rows1_l,
                      num_s, den_s, gsem0, gsem1, ssem0, ssem1):
    half = Df // 2
    cid = lax.axis_index("c")
    sid = lax.axis_index("s")
    w_l = w0_l

    pltpu.sync_copy(as_hbm, as_l)
    pltpu.sync_copy(ad_hbm, ad_l)
    pltpu.sync_copy(src_hbm.at[sid], src_l)
    pltpu.sync_copy(dst_hbm.at[sid], dst_l)

    def _maxstep_a(i, m):
        return jnp.maximum(m, as_l[pl.ds(i * _L, _L)])
    def _maxstep_b(i, m):
        return jnp.maximum(m, ad_l[pl.ds(i * _L, _L)])
    neg = jnp.full((_L,), -3e38, jnp.float32)

    def _lane_max_splat(v16):
        w_l[pl.ds(0, _L)] = v16
        acc = plsc.load_gather(w_l, [jnp.zeros((_L,), jnp.int32)])
        for k in range(1, _L):
            acc = jnp.maximum(
                acc, plsc.load_gather(w_l, [jnp.full((_L,), k, jnp.int32)]))
        return acc

    masv = _lane_max_splat(lax.fori_loop(0, _NPAD // _L, _maxstep_a, neg))
    madv = _lane_max_splat(lax.fori_loop(0, _NPAD // _L, _maxstep_b, neg))
    msum = masv + madv
    cbound = jnp.maximum(msum, 0.2 * msum)

    zero16 = jnp.zeros((_L,), jnp.float32)
    def _zrow(i, _):
        for f in range(half // _L):
            rows0_l[i, pl.ds(f * _L, _L)] = zero16
            rows1_l[i, pl.ds(f * _L, _L)] = zero16
        return 0
    lax.fori_loop(0, _CHUNK, _zrow, 0)
    for k in range(_CHUNK // _L):
        w0_l[pl.ds(k * _L, _L)] = zero16
        w1_l[pl.ds(k * _L, _L)] = zero16
    base = sid * _STRIPE
    for t in range(_STRIPE // _CHUNK):
        pltpu.sync_copy(rows0_l, num_s.at[pl.ds(base + t * _CHUNK, _CHUNK)])
        pltpu.sync_copy(w0_l, den_s.at[pl.ds(base + t * _CHUNK, _CHUNK)])
    plsc.subcore_barrier()

    def _weights(j, wbuf):
        @plsc.parallel_loop(0, _CHUNK // _L, unroll=4)
        def _wgrp(k):
            si = src_l[j, pl.ds(k * _L, _L)]
            di = dst_l[j, pl.ds(k * _L, _L)]
            s = (plsc.load_gather(as_l, [si]) +
                 plsc.load_gather(ad_l, [di]))
            e = jnp.maximum(s, 0.2 * s)
            wbuf[pl.ds(k * _L, _L)] = jnp.exp(e - cbound)

    def _scale_rows(wbuf, rows):
        @plsc.parallel_loop(0, _CHUNK, unroll=8)
        def _scale(i):
            wv = plsc.load_gather(
                wbuf, [jnp.broadcast_to(i, (_L,)).astype(jnp.int32)])
            for f in range(half // _L):
                sl = pl.ds(f * _L, _L)
                rows[i, sl] = rows[i, sl] * wv

    def _issue_gather(j, rows, gsem):
        return pltpu.async_copy(hs_hbm.at[cid].at[src_l.at[j]], rows, gsem)

    def _issue_scatter(j, wbuf, rows, ssem):
        pltpu.async_copy(rows, num_s.at[dst_l.at[j]], ssem, add=True)
        pltpu.async_copy(wbuf, den_s.at[dst_l.at[j]], ssem, add=True)

    def _drain_gather(rows, gsem):
        pltpu.make_async_copy(hs_hbm.at[cid, pl.ds(0, _CHUNK)],
                              rows, gsem).wait()

    def _drain_scatter(wbuf, rows, ssem):
        pltpu.make_async_copy(hs_hbm.at[cid, pl.ds(0, _CHUNK)],
                              rows, ssem).wait()
        pltpu.make_async_copy(as_hbm.at[pl.ds(0, _CHUNK)],
                              wbuf, ssem).wait()

    _issue_scatter(0, w0_l, rows0_l, ssem0)
    _issue_scatter(0, w1_l, rows1_l, ssem1)
    _drain_scatter(w0_l, rows0_l, ssem0)
    _issue_gather(0, rows0_l, gsem0)

    def _pair(k, _):
        a = 2 * k
        b = a + 1
        _drain_scatter(w1_l, rows1_l, ssem1)
        _issue_gather(b, rows1_l, gsem1)
        _weights(a, w0_l)
        _drain_gather(rows0_l, gsem0)
        _scale_rows(w0_l, rows0_l)
        _issue_scatter(a, w0_l, rows0_l, ssem0)
        _weights(b, w1_l)
        _drain_gather(rows1_l, gsem1)
        _scale_rows(w1_l, rows1_l)
        _drain_scatter(w0_l, rows0_l, ssem0)
        _issue_gather(jnp.minimum(a + 2, _NCHUNK - 1), rows0_l, gsem0)
        _issue_scatter(b, w1_l, rows1_l, ssem1)
        return 0
    lax.fori_loop(0, _NCHUNK // 2, _pair, 0)
    _drain_gather(rows0_l, gsem0)
    _drain_scatter(w1_l, rows1_l, ssem1)
    plsc.subcore_barrier()

    pltpu.sync_copy(num_s.at[pl.ds(base, _STRIPE)],
                    num_out.at[cid, pl.ds(base, _STRIPE)])
    @pl.when(cid == 0)
    def _():
        pltpu.sync_copy(den_s.at[pl.ds(base, _STRIPE)],
                        den_out.at[pl.ds(base, _STRIPE)])


def _make_edge_kernel(Df):
    half = Df // 2
    mesh = plsc.VectorSubcoreMesh(core_axis_name="c", subcore_axis_name="s")
    return pl.kernel(
        functools.partial(_edge_kernel_body, Df),
        out_type=(jax.ShapeDtypeStruct((_NC, _NPAD, half), jnp.float32),
                  jax.ShapeDtypeStruct((_NPAD,), jnp.float32)),
        mesh=mesh,
        scratch_types=(
            pltpu.VMEM((_NPAD,), jnp.float32),
            pltpu.VMEM((_NPAD,), jnp.float32),
            pltpu.VMEM((_NCHUNK, _CHUNK), jnp.int32),
            pltpu.VMEM((_NCHUNK, _CHUNK), jnp.int32),
            pltpu.VMEM((_CHUNK,), jnp.float32),
            pltpu.VMEM((_CHUNK,), jnp.float32),
            pltpu.VMEM((_CHUNK, half), jnp.float32),
            pltpu.VMEM((_CHUNK, half), jnp.float32),
            pltpu.VMEM_SHARED((_NPAD, half), jnp.float32),
            pltpu.VMEM_SHARED((_NPAD,), jnp.float32),
            pltpu.SemaphoreType.DMA,
            pltpu.SemaphoreType.DMA,
            pltpu.SemaphoreType.DMA,
            pltpu.SemaphoreType.DMA,
        ),
        compiler_params=pltpu.CompilerParams(
            needs_layout_passes=False, use_tc_tiling_on_sc=False),
        name=f"gat_edge_sc_{Df}",
    )


def _col_from_tile8(v8):
    n = _ROWBLK
    si = lax.broadcasted_iota(jnp.int32, (n, 8), 0) // _G
    sj = lax.broadcasted_iota(jnp.int32, (n, 8), 1)
    p = (si == sj).astype(jnp.float32)
    li = lax.broadcasted_iota(jnp.int32, (n, _G), 0) % _G
    lj = lax.broadcasted_iota(jnp.int32, (n, _G), 1)
    lmask = (li == lj).astype(jnp.float32)
    rep = lax.dot_general(p, v8, (((1,), (0,)), ((), ())),
                          preferred_element_type=jnp.float32)
    return (rep * lmask).sum(-1, keepdims=True)

def _tc_a_body(x_ref, w_ref, asr, adr, h_ref, sa_ref, da_ref):
    h = jnp.dot(x_ref[...], w_ref[...], preferred_element_type=jnp.float32)
    h_ref[0] = h[:, :_HID // 2]
    h_ref[1] = h[:, _HID // 2:]
    sa_ref[...] = (h * asr[...]).sum(-1, keepdims=True).reshape(8, _G)
    da_ref[...] = (h * adr[...]).sum(-1, keepdims=True).reshape(8, _G)


def _tc_b_body(num_ref, den_ref, b1_ref, w2_ref, asr, adr,
               h2_ref, sa_ref, da_ref):
    d = _col_from_tile8(den_ref[...]) + 1e-30
    nsum = jnp.concatenate([num_ref[0], num_ref[1]], axis=-1)
    h = jax.nn.relu(nsum / d + b1_ref[...])
    h2 = jnp.dot(h, w2_ref[...], preferred_element_type=jnp.float32)
    h2_ref[0] = h2[:, :_OUT // 2]
    h2_ref[1] = h2[:, _OUT // 2:]
    sa_ref[...] = (h2 * asr[...]).sum(-1, keepdims=True).reshape(8, _G)
    da_ref[...] = (h2 * adr[...]).sum(-1, keepdims=True).reshape(8, _G)


def _tc_c_body(num_ref, den_ref, b2_ref, bat_ref, lw_ref, lb_ref,
               res_ref, acc_ref):
    i = pl.program_id(0)
    @pl.when(i == 0)
    def _():
        acc_ref[...] = jnp.zeros_like(acc_ref)
    d = _col_from_tile8(den_ref[...]) + 1e-30
    nsum = jnp.concatenate([num_ref[0], num_ref[1]], axis=-1)
    o = nsum / d + b2_ref[...]
    z = o - o.max(-1, keepdims=True)
    ls = z - jnp.log(jnp.exp(z).sum(-1, keepdims=True))
    lsa = jnp.concatenate([ls, jnp.ones_like(ls)], axis=-1)
    bat_col = _col_from_tile8(bat_ref[...].astype(jnp.float32))
    boh = (bat_col == lax.broadcasted_iota(
        jnp.int32, (1, _G), 1).astype(jnp.float32)).astype(jnp.float32)
    acc_ref[...] += lax.dot_general(
        boh, lsa, (((0,), (0,)), ((), ())),
        preferred_element_type=jnp.float32)
    @pl.when(i == pl.num_programs(0) - 1)
    def _():
        acc = acc_ref[...]
        pm = acc[:, :_OUT] / jnp.maximum(acc[:, _OUT:_OUT + 1], 1.0)
        res_ref[...] = jnp.dot(pm, lw_ref[...],
                               preferred_element_type=jnp.float32) + lb_ref[...]


_NB = _NPAD // _ROWBLK


def _tc_a(xp, W1, a1s, a1d):
    return pl.pallas_call(
        _tc_a_body,
        grid=(_NB,),
        in_specs=[
            pl.BlockSpec((_ROWBLK, _D), lambda i: (i, 0)),
            pl.BlockSpec((_D, _HID), lambda i: (0, 0)),
            pl.BlockSpec((1, _HID), lambda i: (0, 0)),
            pl.BlockSpec((1, _HID), lambda i: (0, 0)),
        ],
        out_specs=[
            pl.BlockSpec((_NC, _ROWBLK, _HID // 2), lambda i: (0, i, 0)),
            pl.BlockSpec((8, _G), lambda i: (i, 0)),
            pl.BlockSpec((8, _G), lambda i: (i, 0)),
        ],
        out_shape=[
            jax.ShapeDtypeStruct((_NC, _NPAD, _HID // 2), jnp.float32),
            jax.ShapeDtypeStruct((_NB * 8, _G), jnp.float32),
            jax.ShapeDtypeStruct((_NB * 8, _G), jnp.float32),
        ],
    )(xp, W1, a1s, a1d)


def _tc_b(num1, den1, b1, W2, a2s, a2d):
    return pl.pallas_call(
        _tc_b_body,
        grid=(_NB,),
        in_specs=[
            pl.BlockSpec((_NC, _ROWBLK, _HID // 2), lambda i: (0, i, 0)),
            pl.BlockSpec((8, _G), lambda i: (i, 0)),
            pl.BlockSpec((1, _HID), lambda i: (0, 0)),
            pl.BlockSpec((_HID, _OUT), lambda i: (0, 0)),
            pl.BlockSpec((1, _OUT), lambda i: (0, 0)),
            pl.BlockSpec((1, _OUT), lambda i: (0, 0)),
        ],
        out_specs=[
            pl.BlockSpec((_NC, _ROWBLK, _OUT // 2), lambda i: (0, i, 0)),
            pl.BlockSpec((8, _G), lambda i: (i, 0)),
            pl.BlockSpec((8, _G), lambda i: (i, 0)),
        ],
        out_shape=[
            jax.ShapeDtypeStruct((_NC, _NPAD, _OUT // 2), jnp.float32),
            jax.ShapeDtypeStruct((_NB * 8, _G), jnp.float32),
            jax.ShapeDtypeStruct((_NB * 8, _G), jnp.float32),
        ],
    )(num1, den1, b1, W2, a2s, a2d)


def _tc_c(num2, den2, b2, bat2d, lwp, lbp):
    return pl.pallas_call(
        _tc_c_body,
        grid=(_NB,),
        in_specs=[
            pl.BlockSpec((_NC, _ROWBLK, _OUT // 2), lambda i: (0, i, 0)),
            pl.BlockSpec((8, _G), lambda i: (i, 0)),
            pl.BlockSpec((1, _OUT), lambda i: (0, 0)),
            pl.BlockSpec((8, _G), lambda i: (i, 0)),
            pl.BlockSpec((_OUT, _G), lambda i: (0, 0)),
            pl.BlockSpec((1, _G), lambda i: (0, 0)),
        ],
        out_specs=pl.BlockSpec((_G, _G), lambda i: (0, 0)),
        out_shape=jax.ShapeDtypeStruct((_G, _G), jnp.float32),
        scratch_shapes=[pltpu.VMEM((_G, _G), jnp.float32)],
    )(num2, den2, b2, bat2d, lwp, lbp)


def kernel(x, adj, batch, W1, a1_src, a1_dst, b1, W2, a2_src, a2_dst, b2,
           linW, linb):
    xp = jnp.pad(x, ((0, _NPAD - _N), (0, 0)))
    loops = jnp.arange(_N, dtype=jnp.int32)
    src_all = jnp.concatenate(
        [adj[0], loops, jnp.zeros((_EPAD - _ETOT,), jnp.int32)])
    dst_all = jnp.concatenate(
        [adj[1], loops, jnp.full((_EPAD - _ETOT,), _N, jnp.int32)])
    srcg = src_all.reshape(_NS, _NCHUNK, _CHUNK)
    dstg = dst_all.reshape(_NS, _NCHUNK, _CHUNK)
    batch_pad = jnp.concatenate(
        [batch, jnp.full((_NPAD - _N,), _G, jnp.int32)])
    bat2d = batch_pad.reshape(_NB * 8, _G)
    lwp = jnp.pad(linW, ((0, 0), (0, _G - 1)))
    lbp = jnp.broadcast_to(linb.reshape(1, 1), (1, _G))
    b1r = b1.reshape(1, _HID)
    b2r = b2.reshape(1, _OUT)

    hs1, as1, ad1 = _tc_a(xp, W1, a1_src, a1_dst)
    ek1 = _make_edge_kernel(_HID)
    num1p, den1 = ek1(hs1, as1.reshape(_NPAD), ad1.reshape(_NPAD),
                      srcg, dstg)

    hs2, as2, ad2 = _tc_b(num1p, den1.reshape(_NB * 8, _G), b1r, W2,
                          a2_src, a2_dst)
    ek2 = _make_edge_kernel(_OUT)
    num2p, den2 = ek2(hs2, as2.reshape(_NPAD), ad2.reshape(_NPAD),
                      srcg, dstg)

    res = _tc_c(num2p, den2.reshape(_NB * 8, _G), b2r, bat2d, lwp, lbp)
    return res[:, :1]
